# Initial kernel scaffold; baseline (speedup 1.0000x reference)
import functools
import jax
import jax.numpy as jnp
from jax import lax
from jax.experimental import pallas as pl
from jax.experimental.pallas import tpu as pltpu
from jax.experimental.pallas import tpu_sc as plsc

N_NODES = 10000
N_EDGES = 160000
D_IN = 256
D_OUT = 256
DH = 128

NC = 2
NS = 16
CHUNK = 128
EDGES_PER_TILE = N_EDGES // NS
NFULL = EDGES_PER_TILE // CHUNK
TAIL = EDGES_PER_TILE - NFULL * CHUNK

ROWS_PER_TILE = N_NODES // NS
ZCHUNK = 125

_mesh = plsc.VectorSubcoreMesh(core_axis_name="c", subcore_axis_name="s")


@functools.partial(
    pl.kernel,
    out_type=jax.ShapeDtypeStruct((NC, N_NODES, DH), jnp.float32),
    mesh=_mesh,
    scratch_types=[
        pltpu.VMEM((CHUNK,), jnp.int32),
        pltpu.VMEM((CHUNK,), jnp.int32),
        pltpu.VMEM((CHUNK, DH), jnp.float32),
        pltpu.VMEM((TAIL,), jnp.int32),
        pltpu.VMEM((TAIL,), jnp.int32),
        pltpu.VMEM((TAIL, DH), jnp.float32),
        pltpu.VMEM_SHARED((N_NODES, DH), jnp.float32),
        pltpu.SemaphoreType.DMA,
    ],
)
def _aggregate(x2_hbm, src_hbm, dst_hbm, out_hbm,
               src_v, dst_v, rows_v, src_t, dst_t, rows_t, h_sh, sem):
    c = lax.axis_index("c")
    s = lax.axis_index("s")

    def zrow(i, carry):
        for j in range(DH // 16):
            rows_v[i, pl.ds(j * 16, 16)] = jnp.zeros((16,), jnp.float32)
        return carry
    lax.fori_loop(0, ZCHUNK, zrow, 0)
    for k in range(N_NODES // (NS * ZCHUNK)):
        pltpu.sync_copy(
            rows_v.at[pl.ds(0, ZCHUNK)],
            h_sh.at[pl.ds(s * ROWS_PER_TILE + k * ZCHUNK, ZCHUNK)],
        )
    plsc.subcore_barrier()

    ebase = s * EDGES_PER_TILE

    def chunk_body(i, carry):
        base = ebase + i * CHUNK
        pltpu.sync_copy(src_hbm.at[pl.ds(base, CHUNK)], src_v)
        pltpu.sync_copy(dst_hbm.at[pl.ds(base, CHUNK)], dst_v)
        pltpu.async_copy(x2_hbm.at[c].at[src_v], rows_v, sem).wait()
        pltpu.sync_copy(rows_v, h_sh.at[dst_v], add=True)
        return carry
    lax.fori_loop(0, NFULL, chunk_body, 0)

    tbase = ebase + NFULL * CHUNK
    pltpu.sync_copy(src_hbm.at[pl.ds(tbase, TAIL)], src_t)
    pltpu.sync_copy(dst_hbm.at[pl.ds(tbase, TAIL)], dst_t)
    pltpu.async_copy(x2_hbm.at[c].at[src_t], rows_t, sem).wait()
    pltpu.sync_copy(rows_t, h_sh.at[dst_t], add=True)

    plsc.subcore_barrier()

    pltpu.sync_copy(
        h_sh.at[pl.ds(s * ROWS_PER_TILE, ROWS_PER_TILE)],
        out_hbm.at[c].at[pl.ds(s * ROWS_PER_TILE, ROWS_PER_TILE)],
    )


_BLK = 1000


def _linear_body(h0_ref, h1_ref, w0_ref, w1_ref, b_ref, out_ref):
    dn = (((1,), (1,)), ((), ()))
    acc = lax.dot_general(h0_ref[...], w0_ref[...], dn,
                          preferred_element_type=jnp.float32)
    acc = acc + lax.dot_general(h1_ref[...], w1_ref[...], dn,
                                preferred_element_type=jnp.float32)
    out_ref[...] = acc + b_ref[...]


def _linear(h0, h1, w0, w1, b2):
    return pl.pallas_call(
        _linear_body,
        grid=(N_NODES // _BLK,),
        in_specs=[
            pl.BlockSpec((_BLK, DH), lambda i: (i, 0)),
            pl.BlockSpec((_BLK, DH), lambda i: (i, 0)),
            pl.BlockSpec((D_OUT, DH), lambda i: (0, 0)),
            pl.BlockSpec((D_OUT, DH), lambda i: (0, 0)),
            pl.BlockSpec((1, D_OUT), lambda i: (0, 0)),
        ],
        out_specs=pl.BlockSpec((_BLK, D_OUT), lambda i: (i, 0)),
        out_shape=jax.ShapeDtypeStruct((N_NODES, D_OUT), jnp.float32),
    )(h0, h1, w0, w1, b2)


@jax.jit
def kernel(x, edge_index, W, b):
    src = edge_index[0].astype(jnp.int32)
    dst = edge_index[1].astype(jnp.int32)
    x2 = x.reshape(N_NODES, NC, DH).transpose(1, 0, 2)
    h2 = _aggregate(x2, src, dst)
    return _linear(h2[0], h2[1], W[:, :DH], W[:, DH:], b.reshape(1, D_OUT))


# speedup vs baseline: 4.3623x; 4.3623x over previous
import functools
import jax
import jax.numpy as jnp
from jax import lax
from jax.experimental import pallas as pl
from jax.experimental.pallas import tpu as pltpu
from jax.experimental.pallas import tpu_sc as plsc

N_NODES = 10000
N_EDGES = 160000
D_IN = 256
D_OUT = 256
DH = 128

NC = 2
NS = 16
CHUNK = 128
EDGES_PER_TILE = N_EDGES // NS
NFULL = EDGES_PER_TILE // CHUNK
TAIL = EDGES_PER_TILE - NFULL * CHUNK

ROWS_PER_TILE = (N_NODES // NS) // 8 * 8
REM_ROWS = N_NODES - NS * ROWS_PER_TILE
ZCHUNK = 104

_mesh = plsc.VectorSubcoreMesh(core_axis_name="c", subcore_axis_name="s",
                               num_cores=NC, num_subcores=NS)


@functools.partial(
    pl.kernel,
    out_type=jax.ShapeDtypeStruct((NC, N_NODES, DH), jnp.float32),
    mesh=_mesh,
    scratch_types=[
        pltpu.VMEM((CHUNK,), jnp.int32),
        pltpu.VMEM((CHUNK,), jnp.int32),
        pltpu.VMEM((CHUNK, DH), jnp.float32),
        pltpu.VMEM((TAIL,), jnp.int32),
        pltpu.VMEM((TAIL,), jnp.int32),
        pltpu.VMEM((TAIL, DH), jnp.float32),
        pltpu.VMEM_SHARED((N_NODES, DH), jnp.float32),
        pltpu.SemaphoreType.DMA,
    ],
)
def _aggregate(x2_hbm, src_hbm, dst_hbm, out_hbm,
               src_v, dst_v, rows_v, src_t, dst_t, rows_t, h_sh, sem):
    c = lax.axis_index("c")
    s = lax.axis_index("s")

    def zrow(i, carry):
        for j in range(DH // 16):
            rows_v[i, pl.ds(j * 16, 16)] = jnp.zeros((16,), jnp.float32)
        return carry
    lax.fori_loop(0, ZCHUNK, zrow, 0)
    for k in range(ROWS_PER_TILE // ZCHUNK):
        pltpu.sync_copy(
            rows_v.at[pl.ds(0, ZCHUNK)],
            h_sh.at[pl.ds(s * ROWS_PER_TILE + k * ZCHUNK, ZCHUNK)],
        )

    @pl.when(s == NS - 1)
    def _():
        pltpu.sync_copy(rows_v.at[pl.ds(0, REM_ROWS)],
                        h_sh.at[pl.ds(NS * ROWS_PER_TILE, REM_ROWS)])
    plsc.subcore_barrier()

    ebase = s * EDGES_PER_TILE

    def chunk_body(i, carry):
        base = ebase + i * CHUNK
        pltpu.sync_copy(src_hbm.at[pl.ds(base, CHUNK)], src_v)
        pltpu.sync_copy(dst_hbm.at[pl.ds(base, CHUNK)], dst_v)
        pltpu.async_copy(x2_hbm.at[c].at[src_v], rows_v, sem).wait()
        pltpu.sync_copy(rows_v, h_sh.at[dst_v], add=True)
        return carry
    lax.fori_loop(0, NFULL, chunk_body, 0)

    tbase = ebase + NFULL * CHUNK
    pltpu.sync_copy(src_hbm.at[pl.ds(tbase, TAIL)], src_t)
    pltpu.sync_copy(dst_hbm.at[pl.ds(tbase, TAIL)], dst_t)
    pltpu.async_copy(x2_hbm.at[c].at[src_t], rows_t, sem).wait()
    pltpu.sync_copy(rows_t, h_sh.at[dst_t], add=True)

    plsc.subcore_barrier()

    pltpu.sync_copy(
        h_sh.at[pl.ds(s * ROWS_PER_TILE, ROWS_PER_TILE)],
        out_hbm.at[c].at[pl.ds(s * ROWS_PER_TILE, ROWS_PER_TILE)],
    )

    @pl.when(s == NS - 1)
    def _():
        pltpu.sync_copy(
            h_sh.at[pl.ds(NS * ROWS_PER_TILE, REM_ROWS)],
            out_hbm.at[c].at[pl.ds(NS * ROWS_PER_TILE, REM_ROWS)],
        )


_BLK = 1000


def _linear_body(h0_ref, h1_ref, w0_ref, w1_ref, b_ref, out_ref):
    dn = (((1,), (1,)), ((), ()))
    acc = lax.dot_general(h0_ref[...], w0_ref[...], dn,
                          preferred_element_type=jnp.float32)
    acc = acc + lax.dot_general(h1_ref[...], w1_ref[...], dn,
                                preferred_element_type=jnp.float32)
    out_ref[...] = acc + b_ref[...]


def _linear(h0, h1, w0, w1, b2):
    return pl.pallas_call(
        _linear_body,
        grid=(N_NODES // _BLK,),
        in_specs=[
            pl.BlockSpec((_BLK, DH), lambda i: (i, 0)),
            pl.BlockSpec((_BLK, DH), lambda i: (i, 0)),
            pl.BlockSpec((D_OUT, DH), lambda i: (0, 0)),
            pl.BlockSpec((D_OUT, DH), lambda i: (0, 0)),
            pl.BlockSpec((1, D_OUT), lambda i: (0, 0)),
        ],
        out_specs=pl.BlockSpec((_BLK, D_OUT), lambda i: (i, 0)),
        out_shape=jax.ShapeDtypeStruct((N_NODES, D_OUT), jnp.float32),
    )(h0, h1, w0, w1, b2)


@jax.jit
def kernel(x, edge_index, W, b):
    src = edge_index[0].astype(jnp.int32)
    dst = edge_index[1].astype(jnp.int32)
    x2 = x.reshape(N_NODES, NC, DH).transpose(1, 0, 2)
    h2 = _aggregate(x2, src, dst)
    return _linear(h2[0], h2[1], W[:, :DH], W[:, DH:], b.reshape(1, D_OUT))
